# baseline (speedup 1.0000x reference)
import jax
import jax.numpy as jnp
from jax.experimental import pallas as pl
from jax.experimental.pallas import tpu as pltpu

G = 50


def _fused_kernel(x_ref, w_ref, adj_ref, mu_ref, h_ref, s_scr, h_scr, ht_scr):
    t = pl.program_id(0)
    bm = h_ref.shape[0]

    @pl.when(t == 0)
    def _():
        s_scr[...] = jnp.dot(x_ref[...], w_ref[...],
                             preferred_element_type=jnp.float32)

    @pl.when(t < G)
    def _():
        hblk = jnp.dot(adj_ref[...], s_scr[...],
                       preferred_element_type=jnp.float32)
        h_ref[...] = hblk
        h_scr[pl.ds(t * bm, bm), :] = hblk

    @pl.when(t == G)
    def _():
        ht_scr[...] = h_scr[...].T

    @pl.when(t >= G)
    def _():
        j = t - G
        hi = h_scr[pl.ds(j * bm, bm), :]
        prod = jnp.dot(hi, ht_scr[...], preferred_element_type=jnp.float32)
        mu_ref[...] = jnp.maximum(prod, 0.0)


def kernel(x, adj, W):
    B, N, F = x.shape
    D = W.shape[1]
    x2 = x.reshape(N, F)
    adj2 = adj.reshape(N, N)

    mu, h = pl.pallas_call(
        _fused_kernel,
        grid=(2 * G,),
        in_specs=[
            pl.BlockSpec((N, F), lambda t: (0, 0)),
            pl.BlockSpec((F, D), lambda t: (0, 0)),
            pl.BlockSpec((N // G, N), lambda t: (jnp.minimum(t, G - 1), 0)),
        ],
        out_specs=[
            pl.BlockSpec((N // G, N), lambda t: (jnp.maximum(t - G, 0), 0)),
            pl.BlockSpec((N // G, D), lambda t: (jnp.minimum(t, G - 1), 0)),
        ],
        out_shape=[
            jax.ShapeDtypeStruct((N, N), jnp.float32),
            jax.ShapeDtypeStruct((N, D), jnp.float32),
        ],
        scratch_shapes=[
            pltpu.VMEM((N, D), jnp.float32),
            pltpu.VMEM((N, D), jnp.float32),
            pltpu.VMEM((D, N), jnp.float32),
        ],
    )(x2, W, adj2)
    return (mu.reshape(B, N, N), h.reshape(B, N, D))


# speedup vs baseline: 1.0063x; 1.0063x over previous
import jax
import jax.numpy as jnp
from jax.experimental import pallas as pl
from jax.experimental.pallas import tpu as pltpu

G = 50


def _fused_kernel(x_ref, w_ref, adj_ref, mu_ref, h_ref, s_scr, h_scr, ht_scr):
    t = pl.program_id(0)
    bm = h_ref.shape[0]

    @pl.when(t == 0)
    def _():
        s_scr[...] = jnp.dot(x_ref[...], w_ref[...],
                             preferred_element_type=jnp.float32)

    @pl.when(t < G)
    def _():
        hblk = jnp.dot(adj_ref[...], s_scr[...],
                       preferred_element_type=jnp.float32)
        h_ref[...] = hblk
        h_scr[pl.ds(t * bm, bm), :] = hblk

    @pl.when(t == G)
    def _():
        hi = h_scr[pl.ds(0, bm), :]
        prod = jax.lax.dot_general(
            hi, h_scr[...],
            (((1,), (1,)), ((), ())),
            preferred_element_type=jnp.float32)
        mu_ref[...] = jnp.maximum(prod, 0.0)
        ht_scr[...] = h_scr[...].T

    @pl.when(t > G)
    def _():
        j = t - G
        hi = h_scr[pl.ds(j * bm, bm), :]
        prod = jnp.dot(hi, ht_scr[...], preferred_element_type=jnp.float32)
        mu_ref[...] = jnp.maximum(prod, 0.0)


def kernel(x, adj, W):
    B, N, F = x.shape
    D = W.shape[1]
    x2 = x.reshape(N, F)
    adj2 = adj.reshape(N, N)

    mu, h = pl.pallas_call(
        _fused_kernel,
        grid=(2 * G,),
        in_specs=[
            pl.BlockSpec((N, F), lambda t: (0, 0)),
            pl.BlockSpec((F, D), lambda t: (0, 0)),
            pl.BlockSpec((N // G, N), lambda t: (jnp.minimum(t, G - 1), 0)),
        ],
        out_specs=[
            pl.BlockSpec((N // G, N), lambda t: (jnp.maximum(t - G, 0), 0)),
            pl.BlockSpec((N // G, D), lambda t: (jnp.minimum(t, G - 1), 0)),
        ],
        out_shape=[
            jax.ShapeDtypeStruct((N, N), jnp.float32),
            jax.ShapeDtypeStruct((N, D), jnp.float32),
        ],
        scratch_shapes=[
            pltpu.VMEM((N, D), jnp.float32),
            pltpu.VMEM((N, D), jnp.float32),
            pltpu.VMEM((D, N), jnp.float32),
        ],
    )(x2, W, adj2)
    return (mu.reshape(B, N, N), h.reshape(B, N, D))
